# baseline (speedup 1.0000x reference)
import functools

import jax
import jax.numpy as jnp
from jax.experimental import pallas as pl
from jax.experimental.pallas import tpu as pltpu

E = 8
K = 2
H = 1024
F = 1024
T = 2048
BM = 256
TB = T // BM


def _dot_t(a, b):
    return jax.lax.dot_general(a, b, (((1,), (1,)), ((), ())),
                               preferred_element_type=jnp.float32)


def _compute_gate(x, wqkv):
    mix = _dot_t(x, wqkv)
    q = mix[:, 0:E]
    k = mix[:, E:2 * E]
    v = mix[:, 2 * E:3 * E]
    cols = []
    for i in range(E):
        s = q[:, i:i + 1] * k
        m = jnp.max(s, axis=1, keepdims=True)
        p = jnp.exp(s - m)
        p = p / jnp.sum(p, axis=1, keepdims=True)
        cols.append(jnp.sum(p * v, axis=1, keepdims=True))
    logits = jnp.concatenate(cols, axis=1)
    lm = jnp.max(logits, axis=1, keepdims=True)
    pe = jnp.exp(logits - lm)
    probs = pe / jnp.sum(pe, axis=1, keepdims=True)
    idx = jax.lax.broadcasted_iota(jnp.int32, probs.shape, 1)
    m1 = jnp.max(probs, axis=1, keepdims=True)
    i1 = jnp.min(jnp.where(probs == m1, idx, E), axis=1, keepdims=True)
    probs2 = jnp.where(idx == i1, -jnp.inf, probs)
    m2 = jnp.max(probs2, axis=1, keepdims=True)
    i2 = jnp.min(jnp.where(probs2 == m2, idx, E), axis=1, keepdims=True)
    sel = (idx == i1) | (idx == i2)
    return jnp.where(sel, probs, 0.0) / (m1 + m2)


def _moe_kernel(x_ref, wqkv_ref, w1_ref, w3_ref, w2_ref, out_ref, gate_ref):
    e = pl.program_id(0)
    t = pl.program_id(1)
    x = x_ref[pl.ds(t * BM, BM), :]

    @pl.when(e == 0)
    def _router():
        gate_ref[pl.ds(t * BM, BM), :] = _compute_gate(x, wqkv_ref[...])

    xb = x.astype(jnp.bfloat16)
    h1 = _dot_t(xb, w1_ref[0])
    h3 = _dot_t(xb, w3_ref[0])
    h = (h1 * jax.lax.logistic(h1)) * h3
    ye = _dot_t(h.astype(jnp.bfloat16), w2_ref[0])

    grows = gate_ref[pl.ds(t * BM, BM), :]
    idx = jax.lax.broadcasted_iota(jnp.int32, grows.shape, 1)
    g = jnp.sum(jnp.where(idx == e, grows, 0.0), axis=1, keepdims=True)

    @pl.when(e == 0)
    def _init():
        out_ref[pl.ds(t * BM, BM), :] = g * ye

    @pl.when(e > 0)
    def _acc():
        out_ref[pl.ds(t * BM, BM), :] += g * ye


@functools.partial(jax.jit, static_argnames=("interpret",))
def kernel(hidden_states, wqkv, w1, w3, w2, interpret=False):
    return pl.pallas_call(
        _moe_kernel,
        grid=(E, TB),
        in_specs=[
            pl.BlockSpec((T, H), lambda e, t: (0, 0)),
            pl.BlockSpec((3 * E, H), lambda e, t: (0, 0)),
            pl.BlockSpec((1, F, H), lambda e, t: (e, 0, 0)),
            pl.BlockSpec((1, F, H), lambda e, t: (e, 0, 0)),
            pl.BlockSpec((1, H, F), lambda e, t: (e, 0, 0)),
        ],
        out_specs=pl.BlockSpec((T, H), lambda e, t: (0, 0)),
        out_shape=jax.ShapeDtypeStruct((T, H), jnp.float32),
        scratch_shapes=[pltpu.VMEM((T, E), jnp.float32)],
        interpret=interpret,
    )(hidden_states, wqkv, w1.astype(jnp.bfloat16),
      w3.astype(jnp.bfloat16), w2.astype(jnp.bfloat16))


# speedup vs baseline: 1.1911x; 1.1911x over previous
import functools

import jax
import jax.numpy as jnp
from jax.experimental import pallas as pl
from jax.experimental.pallas import tpu as pltpu

E = 8
K = 2
H = 1024
F = 1024
T = 2048
BM = 256
TB = T // BM


def _dot_t(a, b):
    return jax.lax.dot_general(a, b, (((1,), (1,)), ((), ())),
                               preferred_element_type=jnp.float32)


def _compute_gate(x, wqkv):
    mix = _dot_t(x, wqkv)
    q = mix[:, 0:E]
    k = mix[:, E:2 * E]
    v = mix[:, 2 * E:3 * E]
    cols = []
    for i in range(E):
        s = q[:, i:i + 1] * k
        m = jnp.max(s, axis=1, keepdims=True)
        p = jnp.exp(s - m)
        p = p / jnp.sum(p, axis=1, keepdims=True)
        cols.append(jnp.sum(p * v, axis=1, keepdims=True))
    logits = jnp.concatenate(cols, axis=1)
    lm = jnp.max(logits, axis=1, keepdims=True)
    pe = jnp.exp(logits - lm)
    probs = pe / jnp.sum(pe, axis=1, keepdims=True)
    idx = jax.lax.broadcasted_iota(jnp.int32, probs.shape, 1)
    m1 = jnp.max(probs, axis=1, keepdims=True)
    i1 = jnp.min(jnp.where(probs == m1, idx, E), axis=1, keepdims=True)
    probs2 = jnp.where(idx == i1, -jnp.inf, probs)
    m2 = jnp.max(probs2, axis=1, keepdims=True)
    i2 = jnp.min(jnp.where(probs2 == m2, idx, E), axis=1, keepdims=True)
    sel = (idx == i1) | (idx == i2)
    return jnp.where(sel, probs, 0.0) / (m1 + m2)


def _moe_kernel(x_ref, wqkv_ref, w1_ref, w3_ref, w2_ref, out_ref, gate_ref):
    e = pl.program_id(0)
    t = pl.program_id(1)
    x = x_ref[pl.ds(t * BM, BM), :]

    @pl.when(e == 0)
    def _router():
        gate_ref[pl.ds(t * BM, BM), :] = _compute_gate(x, wqkv_ref[...])

    h1 = _dot_t(x, w1_ref[0])
    h3 = _dot_t(x, w3_ref[0])
    h = (h1 * jax.lax.logistic(h1)) * h3
    ye = _dot_t(h, w2_ref[0])

    grows = gate_ref[pl.ds(t * BM, BM), :]
    idx = jax.lax.broadcasted_iota(jnp.int32, grows.shape, 1)
    g = jnp.sum(jnp.where(idx == e, grows, 0.0), axis=1, keepdims=True)

    @pl.when(e == 0)
    def _init():
        out_ref[pl.ds(t * BM, BM), :] = g * ye

    @pl.when(e > 0)
    def _acc():
        out_ref[pl.ds(t * BM, BM), :] += g * ye


@functools.partial(jax.jit, static_argnames=("interpret",))
def kernel(hidden_states, wqkv, w1, w3, w2, interpret=False):
    return pl.pallas_call(
        _moe_kernel,
        grid=(E, TB),
        in_specs=[
            pl.BlockSpec((T, H), lambda e, t: (0, 0)),
            pl.BlockSpec((3 * E, H), lambda e, t: (0, 0)),
            pl.BlockSpec((1, F, H), lambda e, t: (e, 0, 0)),
            pl.BlockSpec((1, F, H), lambda e, t: (e, 0, 0)),
            pl.BlockSpec((1, H, F), lambda e, t: (e, 0, 0)),
        ],
        out_specs=pl.BlockSpec((T, H), lambda e, t: (0, 0)),
        out_shape=jax.ShapeDtypeStruct((T, H), jnp.float32),
        scratch_shapes=[pltpu.VMEM((T, E), jnp.float32)],
        interpret=interpret,
    )(hidden_states, wqkv, w1, w3, w2)
